# baseline (speedup 1.0000x reference)
import functools

import jax
import jax.numpy as jnp
from jax import lax
from jax.experimental import pallas as pl
from jax.experimental.pallas import tpu as pltpu
from jax.experimental.pallas import tpu_sc as plsc

_B, _D, _K, _A, _H = 4096, 256, 1024, 32, 64
_BLK = 512
_NB = _B // _BLK


def _dist_body(x2_ref, cb_ref, idx_ref, loss_ref):
    i = pl.program_id(0)
    x2 = x2_ref[...]
    cb = cb_ref[...]
    xn = 0.25 * jnp.sum(x2 * x2, axis=1, keepdims=True)
    cbn = jnp.sum(cb * cb, axis=1)
    dot2 = lax.dot_general(
        x2, cb, (((1,), (1,)), ((), ())),
        preferred_element_type=jnp.float32,
    )
    dists = (xn + cbn[None, :]) - dot2
    rowmin = jnp.min(dists, axis=1, keepdims=True)
    ids = lax.broadcasted_iota(jnp.int32, dists.shape, 1).astype(jnp.float32)
    idxf = jnp.min(jnp.where(dists == rowmin, ids, float(_K)), axis=1)
    idx_ref[0, 0, :] = idxf.astype(jnp.int32)

    part = jnp.sum(rowmin).reshape(1, 1)

    @pl.when(i == 0)
    def _():
        loss_ref[...] = jnp.zeros((1, 1), jnp.float32)

    loss_ref[...] += part


def _mlp_body(cb_ref, w1_ref, b1_ref, w2_ref, b2_ref, w3_ref, b3_ref, out_ref):
    cb = cb_ref[...]
    h = jnp.tanh(
        lax.dot_general(cb, w1_ref[...], (((1,), (0,)), ((), ())),
                        preferred_element_type=jnp.float32)
        + b1_ref[...])
    h = jnp.tanh(
        lax.dot_general(h, w2_ref[...], (((1,), (0,)), ((), ())),
                        preferred_element_type=jnp.float32)
        + b2_ref[...])
    out_ref[...] = (
        lax.dot_general(h, w3_ref[...], (((1,), (0,)), ((), ())),
                        preferred_element_type=jnp.float32)
        + b3_ref[...])


_AP = 128


def _sc_gather(table, idx):
    info = plsc.get_sparse_core_info()
    nc, ns = info.num_cores, info.num_subcores
    nw = nc * ns
    b_per_w = _B // nw
    mesh = plsc.VectorSubcoreMesh(core_axis_name="c", subcore_axis_name="s")

    @functools.partial(
        pl.kernel,
        mesh=mesh,
        out_type=jax.ShapeDtypeStruct((_B, _AP), jnp.float32),
        scratch_types=[
            pltpu.VMEM((b_per_w,), jnp.int32),
            pltpu.VMEM((b_per_w, _AP), jnp.float32),
            pltpu.SemaphoreType.DMA,
        ],
    )
    def gather_kernel(table_hbm, idx_hbm, out_hbm, idx_v, rows_v, sem):
        wid = lax.axis_index("s") * nc + lax.axis_index("c")
        base = wid * b_per_w
        pltpu.sync_copy(idx_hbm.at[pl.ds(base, b_per_w)], idx_v)
        pltpu.async_copy(table_hbm.at[idx_v], rows_v, sem).wait()
        pltpu.sync_copy(rows_v, out_hbm.at[pl.ds(base, b_per_w)])

    return gather_kernel(table, idx)


def kernel(x, codebook, W1, b1, W2, b2, W3, b3):
    idx3, loss_sum = pl.pallas_call(
        _dist_body,
        grid=(_NB,),
        in_specs=[
            pl.BlockSpec((_BLK, _D), lambda i: (i, 0)),
            pl.BlockSpec((_K, _D), lambda i: (0, 0)),
        ],
        out_specs=[
            pl.BlockSpec((1, 1, _BLK), lambda i: (i, 0, 0)),
            pl.BlockSpec((1, 1), lambda i: (0, 0)),
        ],
        out_shape=[
            jax.ShapeDtypeStruct((_NB, 1, _BLK), jnp.int32),
            jax.ShapeDtypeStruct((1, 1), jnp.float32),
        ],
    )(x + x, codebook)

    W3p = jnp.pad(W3, ((0, 0), (0, _AP - _A)))
    b3p = jnp.pad(b3, (0, _AP - _A))
    table = pl.pallas_call(
        _mlp_body,
        out_shape=jax.ShapeDtypeStruct((_K, _AP), jnp.float32),
    )(codebook, W1, b1.reshape(1, _H), W2, b2.reshape(1, _H),
      W3p, b3p.reshape(1, _AP))

    idx = idx3.reshape(_B)
    dist = _sc_gather(table, idx)[:, :_A]
    commitment_loss = loss_sum[0, 0] / (_B * _D)
    return dist, commitment_loss
